# Initial kernel scaffold; baseline (speedup 1.0000x reference)
import jax
import jax.numpy as jnp
from jax.experimental import pallas as pl


def kernel(x, embd_weight, dense_W, dense_b):
    raise NotImplementedError("write your pallas kernel here")


# speedup vs baseline: 78.1953x; 78.1953x over previous
import functools

import jax
import jax.numpy as jnp
from jax import lax
from jax.experimental import pallas as pl
from jax.experimental.pallas import tpu as pltpu
from jax.experimental.pallas import tpu_sc as plsc

_B, _L = 16384, 200
_N = _B * _L
_V, _D = 1000, 100
_VPAD = 1024

_NC, _NS = 2, 16
_NW = _NC * _NS
_PER_W = _N // _NW
_CHUNK = 12800
_NCHUNK = _PER_W // _CHUNK
_VECS = _CHUNK // 16


def _proj_body(w_ref, dw_ref, b_ref, out_ref):
    out_ref[:, :] = (
        jnp.dot(w_ref[:, :], dw_ref[:, :], preferred_element_type=jnp.float32)
        + b_ref[0, 0]
    )


def _project_table(embd_weight, dense_W, dense_b):
    w_pad = jnp.zeros((_VPAD, _D), jnp.float32).at[:_V, :].set(embd_weight)
    return pl.pallas_call(
        _proj_body,
        out_shape=jax.ShapeDtypeStruct((_VPAD, 1), jnp.float32),
    )(w_pad, dense_W, dense_b.reshape(1, 1))


_sc_mesh = plsc.VectorSubcoreMesh(core_axis_name="c", subcore_axis_name="s")


@functools.partial(
    pl.kernel,
    mesh=_sc_mesh,
    out_type=jax.ShapeDtypeStruct((_N,), jnp.float32),
    scratch_types=[
        pltpu.VMEM((_VPAD,), jnp.float32),
        pltpu.VMEM((_CHUNK,), jnp.int32),
        pltpu.VMEM((_CHUNK,), jnp.float32),
    ],
    compiler_params=pltpu.CompilerParams(
        needs_layout_passes=False,
        use_tc_tiling_on_sc=False,
    ),
)
def _sc_gather(table_hbm, idx_hbm, out_hbm, table_v, idx_v, out_v):
    wid = lax.axis_index("s") * _NC + lax.axis_index("c")
    base = wid * _PER_W
    pltpu.sync_copy(table_hbm, table_v)

    def chunk_body(c, carry):
        off = base + c * _CHUNK
        pltpu.sync_copy(idx_hbm.at[pl.ds(off, _CHUNK)], idx_v)

        def vec_body(i, carry2):
            sl = pl.ds(i * 16, 16)
            out_v[sl] = plsc.load_gather(table_v, [idx_v[sl]])
            return carry2

        lax.fori_loop(0, _VECS, vec_body, 0, unroll=8)
        pltpu.sync_copy(out_v, out_hbm.at[pl.ds(off, _CHUNK)])
        return carry

    lax.fori_loop(0, _NCHUNK, chunk_body, 0)


def kernel(x, embd_weight, dense_W, dense_b):
    table = _project_table(embd_weight, dense_W, dense_b).reshape(_VPAD)
    idx = x.astype(jnp.int32).reshape(_N)
    out = _sc_gather(table, idx)
    return out.reshape(_B, _L, 1)


# speedup vs baseline: 118.0371x; 1.5095x over previous
import functools

import jax
import jax.numpy as jnp
from jax import lax
from jax.experimental import pallas as pl
from jax.experimental.pallas import tpu as pltpu
from jax.experimental.pallas import tpu_sc as plsc

_B, _L = 16384, 200
_N = _B * _L
_V, _D = 1000, 100
_VPAD = 1024

_NC, _NS = 2, 16
_NW = _NC * _NS
_PER_W = _N // _NW
_CHUNK = 25600
_NCHUNK = _PER_W // _CHUNK
_VECS = _CHUNK // 16


def _proj_body(w_ref, dw_ref, b_ref, out_ref):
    out_ref[:, :] = (
        jnp.dot(w_ref[:, :], dw_ref[:, :], preferred_element_type=jnp.float32)
        + b_ref[0, 0]
    )


def _project_table(embd_weight, dense_W, dense_b):
    w_pad = jnp.zeros((_VPAD, _D), jnp.float32).at[:_V, :].set(embd_weight)
    return pl.pallas_call(
        _proj_body,
        out_shape=jax.ShapeDtypeStruct((_VPAD, 1), jnp.float32),
    )(w_pad, dense_W, dense_b.reshape(1, 1))


_sc_mesh = plsc.VectorSubcoreMesh(core_axis_name="c", subcore_axis_name="s")


@functools.partial(
    pl.kernel,
    mesh=_sc_mesh,
    out_type=jax.ShapeDtypeStruct((_N,), jnp.float32),
    scratch_types=[
        pltpu.VMEM((_VPAD,), jnp.float32),
        pltpu.VMEM((2, _CHUNK), jnp.int32),
        pltpu.VMEM((2, _CHUNK), jnp.float32),
        pltpu.SemaphoreType.DMA((2,)),
        pltpu.SemaphoreType.DMA((2,)),
    ],
    compiler_params=pltpu.CompilerParams(
        needs_layout_passes=False,
        use_tc_tiling_on_sc=False,
    ),
)
def _sc_gather(table_hbm, idx_hbm, out_hbm, table_v, idx_v, out_v,
               sem_in, sem_out):
    wid = lax.axis_index("s") * _NC + lax.axis_index("c")
    base = wid * _PER_W
    pltpu.sync_copy(table_hbm, table_v)

    def in_copy(c, b):
        return pltpu.make_async_copy(
            idx_hbm.at[pl.ds(base + c * _CHUNK, _CHUNK)],
            idx_v.at[b], sem_in.at[b])

    def out_copy(c, b):
        return pltpu.make_async_copy(
            out_v.at[b],
            out_hbm.at[pl.ds(base + c * _CHUNK, _CHUNK)], sem_out.at[b])

    in_copy(0, 0).start()
    in_copy(1, 1).start()
    for c in range(_NCHUNK):
        b = c % 2
        in_copy(c, b).wait()
        if c >= 2:
            out_copy(c - 2, b).wait()

        @plsc.parallel_loop(0, _VECS, unroll=8)
        def _gather_step(i):
            sl = pl.ds(i * 16, 16)
            out_v[b, sl] = plsc.load_gather(table_v, [idx_v[b, sl]])

        out_copy(c, b).start()
        if c + 2 < _NCHUNK:
            in_copy(c + 2, b).start()
    out_copy(_NCHUNK - 2, 0).wait()
    out_copy(_NCHUNK - 1, 1).wait()


def kernel(x, embd_weight, dense_W, dense_b):
    table = _project_table(embd_weight, dense_W, dense_b).reshape(_VPAD)
    idx = x.astype(jnp.int32).reshape(_N)
    out = _sc_gather(table, idx)
    return out.reshape(_B, _L, 1)


# speedup vs baseline: 180.3374x; 1.5278x over previous
import functools

import jax
import jax.numpy as jnp
from jax import lax
from jax.experimental import pallas as pl
from jax.experimental.pallas import tpu as pltpu
from jax.experimental.pallas import tpu_sc as plsc

_B, _L = 16384, 200
_N = _B * _L
_V, _D = 1000, 100
_VPAD = 1024

_NC, _NS = 2, 16
_NW = _NC * _NS
_ROWS_W = _B // _NW
_RCHUNK = 64
_NCHUNK = _ROWS_W // _RCHUNK
_COLS = tuple(range(0, 192, 16)) + (184,)


def _proj_body(w_ref, dw_ref, b_ref, out_ref):
    out_ref[:, :] = (
        jnp.dot(w_ref[:, :], dw_ref[:, :], preferred_element_type=jnp.float32)
        + b_ref[0, 0]
    )


def _project_table(embd_weight, dense_W, dense_b):
    w_pad = jnp.zeros((_VPAD, _D), jnp.float32).at[:_V, :].set(embd_weight)
    return pl.pallas_call(
        _proj_body,
        out_shape=jax.ShapeDtypeStruct((_VPAD, 1), jnp.float32),
    )(w_pad, dense_W, dense_b.reshape(1, 1))


_sc_mesh = plsc.VectorSubcoreMesh(core_axis_name="c", subcore_axis_name="s")


@functools.partial(
    pl.kernel,
    mesh=_sc_mesh,
    out_type=jax.ShapeDtypeStruct((_B, _L), jnp.float32),
    scratch_types=[
        pltpu.VMEM((_VPAD,), jnp.float32),
        pltpu.VMEM((2, _RCHUNK, _L), jnp.int32),
        pltpu.VMEM((2, _RCHUNK, _L), jnp.float32),
        pltpu.SemaphoreType.DMA((2,)),
        pltpu.SemaphoreType.DMA((2,)),
    ],
    compiler_params=pltpu.CompilerParams(
        needs_layout_passes=False,
        use_tc_tiling_on_sc=True,
    ),
)
def _sc_gather(table_hbm, idx_hbm, out_hbm, table_v, idx_v, out_v,
               sem_in, sem_out):
    wid = lax.axis_index("s") * _NC + lax.axis_index("c")
    base = wid * _ROWS_W
    pltpu.sync_copy(table_hbm, table_v)

    def in_copy(c, b):
        return pltpu.make_async_copy(
            idx_hbm.at[pl.ds(base + c * _RCHUNK, _RCHUNK), :],
            idx_v.at[b], sem_in.at[b])

    def out_copy(c, b):
        return pltpu.make_async_copy(
            out_v.at[b],
            out_hbm.at[pl.ds(base + c * _RCHUNK, _RCHUNK), :], sem_out.at[b])

    in_copy(0, 0).start()
    in_copy(1, 1).start()
    for c in range(_NCHUNK):
        b = c % 2
        in_copy(c, b).wait()
        if c >= 2:
            out_copy(c - 2, b).wait()

        @plsc.parallel_loop(0, _RCHUNK, unroll=2)
        def _gather_row(r):
            for co in _COLS:
                sl = pl.ds(co, 16)
                out_v[b, r, sl] = plsc.load_gather(table_v, [idx_v[b, r, sl]])

        out_copy(c, b).start()
        if c + 2 < _NCHUNK:
            in_copy(c + 2, b).start()
    out_copy(_NCHUNK - 2, 0).wait()
    out_copy(_NCHUNK - 1, 1).wait()


def kernel(x, embd_weight, dense_W, dense_b):
    table = _project_table(embd_weight, dense_W, dense_b).reshape(_VPAD)
    idx = x.astype(jnp.int32)
    out = _sc_gather(table, idx)
    return out.reshape(_B, _L, 1)


# speedup vs baseline: 281.2876x; 1.5598x over previous
import functools

import jax
import jax.numpy as jnp
from jax import lax
from jax.experimental import pallas as pl
from jax.experimental.pallas import tpu as pltpu
from jax.experimental.pallas import tpu_sc as plsc

_B, _L = 16384, 200
_N = _B * _L
_V, _D = 1000, 100
_VPAD = 1024

_NC, _NS = 2, 16
_NW = _NC * _NS
_CW = _B // _NW
_RB = 8
_NBLK = _L // _RB


def _proj_body(w_ref, dw_ref, b_ref, out_ref):
    out_ref[:, :] = (
        jnp.dot(w_ref[:, :], dw_ref[:, :], preferred_element_type=jnp.float32)
        + b_ref[0, 0]
    )


def _project_table(embd_weight, dense_W, dense_b):
    w_pad = jnp.zeros((_VPAD, _D), jnp.float32).at[:_V, :].set(embd_weight)
    return pl.pallas_call(
        _proj_body,
        out_shape=jax.ShapeDtypeStruct((_VPAD, 1), jnp.float32),
    )(w_pad, dense_W, dense_b.reshape(1, 1))


_sc_mesh = plsc.VectorSubcoreMesh(core_axis_name="c", subcore_axis_name="s")


@functools.partial(
    pl.kernel,
    mesh=_sc_mesh,
    out_type=jax.ShapeDtypeStruct((_L, _B // 1024, 8, 128), jnp.float32),
    scratch_types=[
        pltpu.VMEM((_VPAD,), jnp.float32),
        pltpu.VMEM((2, _RB, _CW), jnp.int32),
        pltpu.VMEM((2, _RB, 4, 128), jnp.float32),
        pltpu.SemaphoreType.DMA((2,)),
        pltpu.SemaphoreType.DMA((2,)),
    ],
    compiler_params=pltpu.CompilerParams(
        needs_layout_passes=False,
        use_tc_tiling_on_sc=True,
    ),
)
def _sc_gather(table_hbm, idx_hbm, out_hbm, table_v, idx_v, out_v,
               sem_in, sem_out):
    wid = lax.axis_index("s") * _NC + lax.axis_index("c")
    c0 = wid * _CW
    k0 = wid // 2
    r0 = (wid % 2) * 4
    pltpu.sync_copy(table_hbm, table_v)

    def in_copy(m, b):
        return pltpu.make_async_copy(
            idx_hbm.at[pl.ds(m * _RB, _RB), pl.ds(c0, _CW)],
            idx_v.at[b], sem_in.at[b])

    def out_copy(m, b):
        return pltpu.make_async_copy(
            out_v.at[b],
            out_hbm.at[pl.ds(m * _RB, _RB), k0, pl.ds(r0, 4), :],
            sem_out.at[b])

    in_copy(0, 0).start()
    in_copy(1, 1).start()

    def block(m, carry):
        b = lax.rem(m, 2)
        in_copy(m, b).wait()

        @pl.when(m >= 2)
        def _():
            out_copy(m - 2, b).wait()

        @plsc.parallel_loop(0, _RB, unroll=2)
        def _gather_row(l):
            for j in range(_CW // 16):
                c = j * 16
                out_v[b, l, c // 128, pl.ds(c % 128, 16)] = plsc.load_gather(
                    table_v, [idx_v[b, l, pl.ds(c, 16)]])

        out_copy(m, b).start()

        @pl.when(m + 2 < _NBLK)
        def _():
            in_copy(m + 2, b).start()

        return carry

    lax.fori_loop(0, _NBLK, block, 0)
    out_copy(_NBLK - 2, (_NBLK - 2) % 2).wait()
    out_copy(_NBLK - 1, (_NBLK - 1) % 2).wait()


def kernel(x, embd_weight, dense_W, dense_b):
    table = _project_table(embd_weight, dense_W, dense_b).reshape(_VPAD)
    idx_t = jnp.swapaxes(x.astype(jnp.int32), 0, 1)
    out4 = _sc_gather(table, idx_t)
    out = jnp.transpose(out4, (1, 2, 3, 0))
    return out.reshape(_B, _L, 1)


# speedup vs baseline: 309.7409x; 1.1012x over previous
import functools

import jax
import jax.numpy as jnp
from jax import lax
from jax.experimental import pallas as pl
from jax.experimental.pallas import tpu as pltpu
from jax.experimental.pallas import tpu_sc as plsc

_B, _L = 16384, 200
_N = _B * _L
_V, _D = 1000, 100
_VPAD = 1024

_NC, _NS = 2, 16
_NW = _NC * _NS
_CW = _B // _NW
_RB = 8
_NBLK = _L // _RB
_NBUF = 4


def _proj_body(w_ref, dw_ref, b_ref, out_ref):
    out_ref[:, :] = (
        jnp.dot(w_ref[:, :], dw_ref[:, :], preferred_element_type=jnp.float32)
        + b_ref[0, 0]
    )


def _project_table(embd_weight, dense_W, dense_b):
    w_pad = jnp.zeros((_VPAD, _D), jnp.float32).at[:_V, :].set(embd_weight)
    return pl.pallas_call(
        _proj_body,
        out_shape=jax.ShapeDtypeStruct((_VPAD, 1), jnp.float32),
    )(w_pad, dense_W, dense_b.reshape(1, 1))


_sc_mesh = plsc.VectorSubcoreMesh(core_axis_name="c", subcore_axis_name="s")


@functools.partial(
    pl.kernel,
    mesh=_sc_mesh,
    out_type=jax.ShapeDtypeStruct((_L, _B // 1024, 8, 128), jnp.float32),
    scratch_types=[
        pltpu.VMEM((_VPAD,), jnp.float32),
        pltpu.VMEM((_NBUF, _RB, _CW), jnp.int32),
        pltpu.VMEM((_NBUF, _RB, 4, 128), jnp.float32),
        pltpu.SemaphoreType.DMA((_NBUF,)),
        pltpu.SemaphoreType.DMA((_NBUF,)),
    ],
    compiler_params=pltpu.CompilerParams(
        needs_layout_passes=False,
        use_tc_tiling_on_sc=True,
    ),
)
def _sc_gather(table_hbm, idx_hbm, out_hbm, table_v, idx_v, out_v,
               sem_in, sem_out):
    wid = lax.axis_index("s") * _NC + lax.axis_index("c")
    c0 = wid * _CW
    k0 = wid // 2
    r0 = (wid % 2) * 4
    pltpu.sync_copy(table_hbm, table_v)

    def in_copy(m, b):
        return pltpu.make_async_copy(
            idx_hbm.at[pl.ds(m * _RB, _RB), pl.ds(c0, _CW)],
            idx_v.at[b], sem_in.at[b])

    def out_copy(m, b):
        return pltpu.make_async_copy(
            out_v.at[b],
            out_hbm.at[pl.ds(m * _RB, _RB), k0, pl.ds(r0, 4), :],
            sem_out.at[b])

    for p in range(_NBUF):
        in_copy(p, p).start()

    def block(m, carry):
        b = lax.rem(m, _NBUF)
        in_copy(m, b).wait()

        @pl.when(m >= _NBUF)
        def _():
            out_copy(m - _NBUF, b).wait()

        @plsc.parallel_loop(0, _RB, unroll=2)
        def _gather_row(l):
            for j in range(_CW // 16):
                c = j * 16
                out_v[b, l, c // 128, pl.ds(c % 128, 16)] = plsc.load_gather(
                    table_v, [idx_v[b, l, pl.ds(c, 16)]])

        out_copy(m, b).start()

        @pl.when(m + _NBUF < _NBLK)
        def _():
            in_copy(m + _NBUF, b).start()

        return carry

    lax.fori_loop(0, _NBLK, block, 0)
    for p in range(_NBUF):
        m = _NBLK - _NBUF + p
        out_copy(m, m % _NBUF).wait()


def kernel(x, embd_weight, dense_W, dense_b):
    table = _project_table(embd_weight, dense_W, dense_b).reshape(_VPAD)
    idx_t = jnp.swapaxes(x.astype(jnp.int32), 0, 1)
    out4 = _sc_gather(table, idx_t)
    out = jnp.transpose(out4, (1, 2, 3, 0))
    return out.reshape(_B, _L, 1)


# speedup vs baseline: 332.3058x; 1.0729x over previous
import functools

import jax
import jax.numpy as jnp
from jax import lax
from jax.experimental import pallas as pl
from jax.experimental.pallas import tpu as pltpu
from jax.experimental.pallas import tpu_sc as plsc

_B, _L = 16384, 200
_N = _B * _L
_V, _D = 1000, 100
_VPAD = 1024

_NC, _NS = 2, 16
_NW = _NC * _NS
_CW = _B // _NW
_RB = 8
_NBLK = _L // _RB
_NBUF = 4

_sc_mesh = plsc.VectorSubcoreMesh(core_axis_name="c", subcore_axis_name="s")


@functools.partial(
    pl.kernel,
    mesh=_sc_mesh,
    out_type=jax.ShapeDtypeStruct((_L, _B // 1024, 8, 128), jnp.float32),
    scratch_types=[
        pltpu.VMEM((_VPAD,), jnp.float32),
        pltpu.VMEM((_D, 128), jnp.float32),
        pltpu.VMEM((_D + 12,), jnp.float32),
        pltpu.VMEM_SHARED((_VPAD,), jnp.float32),
        pltpu.VMEM((_NBUF, _RB, _CW), jnp.int32),
        pltpu.VMEM((_NBUF, _RB, 4, 128), jnp.float32),
        pltpu.SemaphoreType.DMA((_NBUF,)),
        pltpu.SemaphoreType.DMA((_NBUF,)),
    ],
    compiler_params=pltpu.CompilerParams(
        needs_layout_passes=False,
        use_tc_tiling_on_sc=True,
    ),
)
def _sc_kernel(wt_hbm, dwb_hbm, idx_hbm, out_hbm,
               table_v, wblk_v, dw_v, table_sh, idx_v, out_v,
               sem_in, sem_out):
    sid = lax.axis_index("s")
    wid = sid * _NC + lax.axis_index("c")
    c0 = wid * _CW
    k0 = wid // 2
    r0 = (wid % 2) * 4

    def in_copy(m, b):
        return pltpu.make_async_copy(
            idx_hbm.at[pl.ds(m * _RB, _RB), pl.ds(c0, _CW)],
            idx_v.at[b], sem_in.at[b])

    def out_copy(m, b):
        return pltpu.make_async_copy(
            out_v.at[b],
            out_hbm.at[pl.ds(m * _RB, _RB), k0, pl.ds(r0, 4), :],
            sem_out.at[b])

    for p in range(_NBUF):
        in_copy(p, p).start()

    pltpu.sync_copy(dwb_hbm, dw_v)

    @pl.when(sid < 8)
    def _build():
        v0 = sid * 128

        pltpu.sync_copy(wt_hbm.at[:, pl.ds(v0, 128)], wblk_v)

        bias = plsc.load_gather(dw_v, [jnp.full((16,), _D, jnp.int32)])
        acc0 = tuple(bias for _ in range(8))

        def fma(d, acc):
            w = plsc.load_gather(dw_v, [jnp.full((16,), d, jnp.int32)])
            return tuple(
                acc[j] + wblk_v[d, pl.ds(j * 16, 16)] * w for j in range(8))

        acc = lax.fori_loop(0, _D, fma, acc0, unroll=2)
        for j in range(8):
            table_v[pl.ds(j * 16, 16)] = acc[j]
        pltpu.sync_copy(table_v.at[pl.ds(0, 128)], table_sh.at[pl.ds(v0, 128)])

    plsc.subcore_barrier()
    pltpu.sync_copy(table_sh, table_v)

    def block(m, carry):
        b = lax.rem(m, _NBUF)
        in_copy(m, b).wait()

        @pl.when(m >= _NBUF)
        def _():
            out_copy(m - _NBUF, b).wait()

        @plsc.parallel_loop(0, _RB, unroll=2)
        def _gather_row(l):
            for j in range(_CW // 16):
                c = j * 16
                out_v[b, l, c // 128, pl.ds(c % 128, 16)] = plsc.load_gather(
                    table_v, [idx_v[b, l, pl.ds(c, 16)]])

        out_copy(m, b).start()

        @pl.when(m + _NBUF < _NBLK)
        def _():
            in_copy(m + _NBUF, b).start()

        return carry

    lax.fori_loop(0, _NBLK, block, 0)
    for p in range(_NBUF):
        m = _NBLK - _NBUF + p
        out_copy(m, m % _NBUF).wait()


def kernel(x, embd_weight, dense_W, dense_b):
    w_pad = jnp.pad(embd_weight, ((0, _VPAD - _V), (0, 0)))
    wt = jnp.swapaxes(w_pad, 0, 1)
    dwb = jnp.concatenate(
        [dense_W.reshape(_D), dense_b, jnp.zeros((11,), jnp.float32)])
    idx_t = jnp.swapaxes(x.astype(jnp.int32), 0, 1)
    out4 = _sc_kernel(wt, dwb, idx_t)
    out = jnp.transpose(out4, (1, 2, 3, 0))
    return out.reshape(_B, _L, 1)
